# baseline (speedup 1.0000x reference)
import dataclasses
import functools

import jax
import jax.numpy as jnp
from jax import lax
from jax.experimental import pallas as pl
from jax.experimental.pallas import tpu as pltpu
from jax.experimental.pallas import tpu_sc as plsc

NEG = -1e30
F32 = jnp.float32

_SC_PARAMS = pltpu.CompilerParams()
if "needs_layout_passes" in pltpu.CompilerParams.__dataclass_fields__:
    _SC_PARAMS = dataclasses.replace(_SC_PARAMS, needs_layout_passes=False)


def _proj_body(x_ref, wcat_ref, fcat_ref):
    fcat_ref[...] = jnp.dot(x_ref[...], wcat_ref[...],
                            preferred_element_type=F32)


def _k2_body(e_pad, ho, fcat_hbm, src_hbm, dst_hbm, fsum_hbm,
             srcb, dstb, a0, b0, a1, b1, gsem0, gsem1, ssem0, ssem1):
    c = lax.axis_index("c")
    s = lax.axis_index("s")
    wid = s * 2 + c
    nch = e_pad // (32 * 128)
    base = wid * nch

    pltpu.sync_copy(src_hbm.at[pl.ds(base, nch)], srcb)
    pltpu.sync_copy(dst_hbm.at[pl.ds(base, nch)], dstb)

    ab = [(a0, b0, gsem0, ssem0), (a1, b1, gsem1, ssem1)]

    def issue_gather(j, k):
        a_v, b_v, gsem, _ = ab[k]
        pltpu.async_copy(fcat_hbm.at[srcb.at[j]], a_v, gsem)
        pltpu.async_copy(fcat_hbm.at[dstb.at[j]], b_v, gsem)

    def wait_gather(k):
        a_v, b_v, gsem, _ = ab[k]
        pltpu.make_async_copy(fcat_hbm.at[srcb.at[0]], a_v, gsem).wait()
        pltpu.make_async_copy(fcat_hbm.at[dstb.at[0]], b_v, gsem).wait()

    def do_chunk(j, k):
        a_v, b_v, _, ssem = ab[k]
        wait_gather(k)

        @pl.loop(0, 128)
        def _row(i):
            ar = a_v.at[i]
            br = b_v.at[i]
            for q in range(ho // 16):
                ar[pl.ds(q * 16, 16)] = (
                    ar[pl.ds(q * 16, 16)] + br[pl.ds(ho + q * 16, 16)])

        pltpu.async_copy(a_v, fsum_hbm.at[pl.ds((base + j) * 128, 128)], ssem)

    def wait_store(k):
        a_v = ab[k][0]
        ssem = ab[k][3]
        pltpu.make_async_copy(a_v, fsum_hbm.at[pl.ds(0, 128)], ssem).wait()

    issue_gather(0, 0)

    @pl.loop(0, nch // 2)
    def _jj(jj):
        for k in range(2):
            j = 2 * jj + k
            jnext = j + 1
            knext = 1 - k

            @pl.when(jnext < nch)
            def _():
                @pl.when(jnext >= 2)
                def _():
                    wait_store(knext)
                issue_gather(jnext, knext)

            do_chunk(j, k)

    wait_store(0)
    wait_store(1)


def _k3_body(be, e_real, fsum_ref, ef_ref, wf_ref, ablk_ref, mmean_ref,
             bias_ref, re_ref, et_ref, c_ref):
    i = pl.program_id(0)
    f = fsum_ref[...][:, :bias_ref.shape[1]] + jnp.dot(
        ef_ref[...], wf_ref[...], preferred_element_type=F32) + bias_ref[...]
    f = jnp.where(f >= 0, f, 0.01 * f)
    re_ref[...] = jnp.dot(f, mmean_ref[...], preferred_element_type=F32)
    et = lax.dot_general(ablk_ref[...], f, (((0,), (1,)), ((), ())),
                         preferred_element_type=F32)
    ids = i * be + lax.broadcasted_iota(jnp.int32, et.shape, 1)
    et = jnp.where(ids < e_real, et, NEG)
    et_ref[...] = et

    @pl.when(i == 0)
    def _():
        c_ref[...] = jnp.full_like(c_ref[...], NEG)

    c_ref[...] = jnp.maximum(c_ref[...], jnp.max(et))


def _k6_body(e_pad, n_nodes, in_n, sp2,
             nf_hbm, src_hbm, dst_hbm, et_hbm, c_hbm, g_hbm, s_hbm,
             srcc, dstc, pc, c_v, x0, x1, slocb,
             g0, g1, s0, s1, gsp):
    core = lax.axis_index("c")
    tid = lax.axis_index("s")
    per = e_pad // 16
    nch = per // 128
    rows_main = (n_nodes // 128) * 8
    rows_extra = n_nodes - 16 * rows_main
    zrows = rows_main // 6
    nq = in_n // 16

    xb = [x0, x1]
    gsems = [g0, g1]
    ssems = [s0, s1]

    pltpu.sync_copy(c_hbm.at[0, pl.ds(0, 16)], c_v)

    def load_idx(h, j, k):
        base = tid * per + j * 128
        pltpu.sync_copy(src_hbm.at[pl.ds(base, 128)], srcc.at[k])
        pltpu.sync_copy(dst_hbm.at[pl.ds(base, 128)], dstc.at[k])
        pltpu.sync_copy(et_hbm.at[h, pl.ds(base, 128)], pc.at[k])

    def issue_gather(k):
        pltpu.async_copy(nf_hbm.at[srcc.at[k]], xb[k], gsems[k])

    def wait_gather(k):
        pltpu.make_async_copy(nf_hbm.at[srcc.at[k]], xb[k], gsems[k]).wait()

    def issue_scatter(k):
        pltpu.async_copy(xb[k], gsp.at[dstc.at[k]], ssems[k], add=True)

    def wait_scatter(k):
        pltpu.make_async_copy(xb[k], gsp.at[dstc.at[k]], ssems[k]).wait()

    def p_compute(k):
        cc = c_v[...]
        pr = pc.at[k]
        dr = dstc.at[k]
        for q in range(8):
            sl = pl.ds(q * 16, 16)
            pv = jnp.exp(pr[sl] - cc)
            pr[sl] = pv
            plsc.addupdate_scatter(slocb, [dr[sl]], pv)

    def scale_chunk(k):
        x_v = xb[k]
        kvec = jnp.full((16,), k, jnp.int32)

        @pl.loop(0, 128, step=2)
        def _edge(i):
            ps0 = plsc.load_gather(
                pc, [kvec, jnp.full((16,), i, jnp.int32)])
            ps1 = plsc.load_gather(
                pc, [kvec, jnp.full((16,), i + 1, jnp.int32)])
            xr0 = x_v.at[i]
            xr1 = x_v.at[i + 1]
            for q in range(nq):
                sl = pl.ds(q * 16, 16)
                xr0[sl] = xr0[sl] * ps0
            for q in range(nq):
                sl = pl.ds(q * 16, 16)
                xr1[sl] = xr1[sl] * ps1

    for kk in range(2):
        h = core * 2 + kk

        @pl.loop(0, 128)
        def _zx(i):
            xr = x0.at[i]
            for q in range(nq):
                xr[pl.ds(q * 16, 16)] = jnp.zeros((16,), F32)

        for z in range(6):
            pltpu.sync_copy(
                x0.at[pl.ds(0, zrows)],
                gsp.at[pl.ds(tid * rows_main + z * zrows, zrows)])

        @pl.when(tid == 15)
        def _():
            pltpu.sync_copy(x0.at[pl.ds(0, rows_extra)],
                            gsp.at[pl.ds(16 * rows_main, rows_extra)])

        @pl.loop(0, sp2 // 16)
        def _zs(i):
            slocb[pl.ds(i * 16, 16)] = jnp.zeros((16,), F32)

        plsc.subcore_barrier()

        load_idx(h, 0, 0)
        issue_gather(0)

        @pl.loop(0, nch // 2)
        def _jj(jj):
            for k in range(2):
                j = 2 * jj + k
                knext = 1 - k

                wait_gather(k)
                p_compute(k)
                scale_chunk(k)

                @pl.when(j >= 1)
                def _():
                    wait_scatter(knext)

                @pl.when(j + 1 < nch)
                def _():
                    load_idx(h, j + 1, knext)
                    issue_gather(knext)

                issue_scatter(k)

        wait_scatter((nch - 1) % 2)

        plsc.subcore_barrier()
        pltpu.sync_copy(
            gsp.at[pl.ds(tid * rows_main, rows_main)],
            g_hbm.at[h, pl.ds(tid * rows_main, rows_main)])

        @pl.when(tid == 15)
        def _():
            pltpu.sync_copy(gsp.at[pl.ds(16 * rows_main, rows_extra)],
                            g_hbm.at[h, pl.ds(16 * rows_main, rows_extra)])

        pltpu.sync_copy(slocb,
                        s_hbm.at[pl.ds((h * 16 + tid) * sp2, sp2)])
        plsc.subcore_barrier()


def _k7_body(heads, out_n, g_ref, s_ref, wn_ref, rn_ref):
    g = g_ref[...]
    s = jnp.sum(s_ref[...], axis=1)
    wn = wn_ref[...]
    acc = jnp.zeros(rn_ref.shape, F32)
    for h in range(heads):
        sh = s[h]
        inv = jnp.where(sh > 0, 1.0 / sh, 0.0)[:, None]
        acc = acc + jnp.dot(g[h] * inv, wn[:, h * out_n:(h + 1) * out_n],
                            preferred_element_type=F32)
    rn_ref[...] = (1.0 / heads) * acc


def kernel(nfeats, efeats, edge_index, W_ni, W_nj, W_fij, W_node, attn, bias):
    N, IN_N = nfeats.shape
    E, IN_E = efeats.shape
    H = attn.shape[1]
    OUT_E = attn.shape[2]
    OUT_N = W_node.shape[1] // H
    HO = H * OUT_E
    E_pad = ((E + 8191) // 8192) * 8192
    pad = E_pad - E

    src_p = jnp.pad(edge_index[0], (0, pad))
    dst_p = jnp.pad(edge_index[1], (0, pad))
    ef_p = jnp.pad(efeats, ((0, pad), (0, 0)))

    Ablk = (attn[0][:, :, None] * jnp.eye(H, dtype=F32)[:, None, :])
    Ablk = Ablk.reshape(HO, H)
    Ablk = jnp.pad(Ablk, ((0, 0), (0, 8 - H)))
    Mmean = jnp.tile(jnp.eye(OUT_E, dtype=F32), (H, 1)) * (1.0 / H)
    bias2 = bias.reshape(1, HO).astype(F32)

    Wcat = jnp.concatenate([W_ni, W_nj], axis=1)
    BN1 = 2000
    fcat = pl.pallas_call(
        _proj_body,
        grid=(N // BN1,),
        in_specs=[
            pl.BlockSpec((BN1, IN_N), lambda i: (i, 0)),
            pl.BlockSpec((IN_N, 2 * HO), lambda i: (0, 0)),
        ],
        out_specs=pl.BlockSpec((BN1, 2 * HO), lambda i: (i, 0)),
        out_shape=jax.ShapeDtypeStruct((N, 2 * HO), F32),
    )(nfeats, Wcat)

    mesh = plsc.VectorSubcoreMesh(core_axis_name="c", subcore_axis_name="s")
    src2 = src_p.reshape(E_pad // 128, 128)
    dst2 = dst_p.reshape(E_pad // 128, 128)
    nch2 = E_pad // (32 * 128)
    fsum = pl.kernel(
        functools.partial(_k2_body, E_pad, HO),
        out_type=jax.ShapeDtypeStruct((E_pad, 2 * HO), F32),
        mesh=mesh,
        scratch_types=[
            pltpu.VMEM((nch2, 128), jnp.int32),
            pltpu.VMEM((nch2, 128), jnp.int32),
            pltpu.VMEM((128, 2 * HO), F32),
            pltpu.VMEM((128, 2 * HO), F32),
            pltpu.VMEM((128, 2 * HO), F32),
            pltpu.VMEM((128, 2 * HO), F32),
            pltpu.SemaphoreType.DMA,
            pltpu.SemaphoreType.DMA,
            pltpu.SemaphoreType.DMA,
            pltpu.SemaphoreType.DMA,
        ],
    )(fcat, src2, dst2)

    BE = 2048
    re_p, et, Carr = pl.pallas_call(
        functools.partial(_k3_body, BE, E),
        grid=(E_pad // BE,),
        in_specs=[
            pl.BlockSpec((BE, 2 * HO), lambda i: (i, 0)),
            pl.BlockSpec((BE, IN_E), lambda i: (i, 0)),
            pl.BlockSpec((IN_E, HO), lambda i: (0, 0)),
            pl.BlockSpec((HO, 8), lambda i: (0, 0)),
            pl.BlockSpec((HO, OUT_E), lambda i: (0, 0)),
            pl.BlockSpec((1, HO), lambda i: (0, 0)),
        ],
        out_specs=(
            pl.BlockSpec((BE, OUT_E), lambda i: (i, 0)),
            pl.BlockSpec((8, BE), lambda i: (0, i)),
            pl.BlockSpec((8, 128), lambda i: (0, 0)),
        ),
        out_shape=(
            jax.ShapeDtypeStruct((E_pad, OUT_E), F32),
            jax.ShapeDtypeStruct((8, E_pad), F32),
            jax.ShapeDtypeStruct((8, 128), F32),
        ),
    )(fsum, ef_p, W_fij, Ablk, Mmean, bias2)

    SP2 = ((N + 127) // 128) * 128
    g, s = pl.kernel(
        functools.partial(_k6_body, E_pad, N, IN_N, SP2),
        out_type=(
            jax.ShapeDtypeStruct((H, N, IN_N), F32),
            jax.ShapeDtypeStruct((H * 16 * SP2,), F32),
        ),
        mesh=plsc.VectorSubcoreMesh(core_axis_name="c", subcore_axis_name="s"),
        compiler_params=_SC_PARAMS,
        scratch_types=[
            pltpu.VMEM((2, 128), jnp.int32),
            pltpu.VMEM((2, 128), jnp.int32),
            pltpu.VMEM((2, 128), F32),
            pltpu.VMEM((16,), F32),
            pltpu.VMEM((128, IN_N), F32),
            pltpu.VMEM((128, IN_N), F32),
            pltpu.VMEM((SP2,), F32),
            pltpu.SemaphoreType.DMA,
            pltpu.SemaphoreType.DMA,
            pltpu.SemaphoreType.DMA,
            pltpu.SemaphoreType.DMA,
            pltpu.VMEM_SHARED((N, IN_N), F32),
        ],
    )(nfeats, src_p, dst_p, et, Carr)

    s3 = s.reshape(H, 16, SP2)
    BN7 = 2048
    rn = pl.pallas_call(
        functools.partial(_k7_body, H, OUT_N),
        grid=((N + BN7 - 1) // BN7,),
        in_specs=[
            pl.BlockSpec((H, BN7, IN_N), lambda i: (0, i, 0)),
            pl.BlockSpec((H, 16, BN7), lambda i: (0, 0, i)),
            pl.BlockSpec((IN_N, H * OUT_N), lambda i: (0, 0)),
        ],
        out_specs=pl.BlockSpec((BN7, OUT_N), lambda i: (i, 0)),
        out_shape=jax.ShapeDtypeStruct((N, OUT_N), F32),
    )(g, s3, W_node)

    return rn, re_p[:E]


# speedup vs baseline: 1.3225x; 1.3225x over previous
import dataclasses
import functools

import jax
import jax.numpy as jnp
from jax import lax
from jax.experimental import pallas as pl
from jax.experimental.pallas import tpu as pltpu
from jax.experimental.pallas import tpu_sc as plsc

NEG = -1e30
F32 = jnp.float32

_SC_PARAMS = pltpu.CompilerParams()
if "needs_layout_passes" in pltpu.CompilerParams.__dataclass_fields__:
    _SC_PARAMS = dataclasses.replace(_SC_PARAMS, needs_layout_passes=False)


def _proj_body(x_ref, wcat_ref, fcat_ref):
    fcat_ref[...] = jnp.dot(x_ref[...], wcat_ref[...],
                            preferred_element_type=F32)


def _k2_body(e_pad, ho, fcat_hbm, src_hbm, dst_hbm, fsum_hbm,
             si_v, a_v, b_v):
    c = lax.axis_index("c")
    s = lax.axis_index("s")
    wid = s * 2 + c
    per = e_pad // 32
    base0 = wid * per

    @pl.loop(0, per // 128)
    def _chunk(j):
        base = base0 + j * 128
        pltpu.sync_copy(src_hbm.at[pl.ds(base, 128)], si_v)
        pltpu.sync_copy(fcat_hbm.at[si_v], a_v)
        pltpu.sync_copy(dst_hbm.at[pl.ds(base, 128)], si_v)
        pltpu.sync_copy(fcat_hbm.at[si_v], b_v)

        @pl.loop(0, 128)
        def _row(i):
            ar = a_v.at[i]
            br = b_v.at[i]
            for k in range(ho // 16):
                ar[pl.ds(k * 16, 16)] = (
                    ar[pl.ds(k * 16, 16)] + br[pl.ds(ho + k * 16, 16)])

        pltpu.sync_copy(a_v, fsum_hbm.at[pl.ds(base, 128)])


def _k3_body(be, e_real, fsum_ref, ef_ref, wf_ref, ablk_ref, mmean_ref,
             bias_ref, re_ref, et_ref, c_ref):
    i = pl.program_id(0)
    f = fsum_ref[...][:, :bias_ref.shape[1]] + jnp.dot(
        ef_ref[...], wf_ref[...], preferred_element_type=F32) + bias_ref[...]
    f = jnp.where(f >= 0, f, 0.01 * f)
    re_ref[...] = jnp.dot(f, mmean_ref[...], preferred_element_type=F32)
    et = lax.dot_general(ablk_ref[...], f, (((0,), (1,)), ((), ())),
                         preferred_element_type=F32)
    ids = i * be + lax.broadcasted_iota(jnp.int32, et.shape, 1)
    et = jnp.where(ids < e_real, et, NEG)
    et_ref[...] = et

    @pl.when(i == 0)
    def _():
        c_ref[...] = jnp.full_like(c_ref[...], NEG)

    c_ref[...] = jnp.maximum(c_ref[...], jnp.max(et))


def _k6_body(e_pad, n_nodes, in_n, nf_hbm, src_hbm, dst_hbm, et_hbm, c_hbm,
             g_hbm, s_hbm, si_v, di_v, x_v, e_v, p_v, c_v, z_v,
             si2_v, di2_v, x2_v, e2_v, gsem, gsem2, ssem, gsp, ssp):
    core = lax.axis_index("c")
    tid = lax.axis_index("s")
    per = e_pad // 16
    nchunks = per // 128
    sp = ssp.shape[0]
    rows_main = (n_nodes // 128) * 8
    rows_extra = n_nodes - 16 * rows_main
    zrows = rows_main // 6
    swords = sp // 16

    pltpu.sync_copy(c_hbm.at[0, pl.ds(0, 16)], c_v)

    @pl.loop(0, swords // 16)
    def _z2(i):
        z_v[pl.ds(i * 16, 16)] = jnp.zeros((16,), F32)

    for k in range(2):
        h = core * 2 + k
        @pl.loop(0, 128)
        def _zx(i):
            xr = x_v.at[i]
            for q in range(in_n // 16):
                xr[pl.ds(q * 16, 16)] = jnp.zeros((16,), F32)

        for z5 in range(6):
            pltpu.sync_copy(
                x_v.at[pl.ds(0, zrows)],
                gsp.at[pl.ds(tid * rows_main + z5 * zrows, zrows)])

        @pl.when(tid == 15)
        def _():
            pltpu.sync_copy(x_v.at[pl.ds(0, rows_extra)],
                            gsp.at[pl.ds(16 * rows_main, rows_extra)])

        pltpu.sync_copy(z_v, ssp.at[pl.ds(tid * swords, swords)])
        plsc.subcore_barrier()

        base0 = tid * per
        pltpu.sync_copy(src_hbm.at[pl.ds(base0, 128)], si_v)
        pltpu.sync_copy(dst_hbm.at[pl.ds(base0, 128)], di_v.at[0])
        pltpu.sync_copy(et_hbm.at[h, pl.ds(base0, 128)], e_v)
        pltpu.async_copy(nf_hbm.at[si_v], x_v, gsem)

        @pl.loop(0, nchunks // 2)
        def _chunk(jj):
            for (xa, sia, dia, ea, ga, xb_, sib, dib, eb, gb) in (
                    (x_v, si_v, di_v, e_v, gsem, x2_v, si2_v, di2_v, e2_v,
                     gsem2),):
                for step in range(2):
                    if step == 0:
                        xc, sic, dic, ec, gc = xa, sia, dia, ea, ga
                        xn, sin_, din, en, gn = xb_, sib, dib, eb, gb
                        j = 2 * jj
                    else:
                        xc, sic, dic, ec, gc = xb_, sib, dib, eb, gb
                        xn, sin_, din, en, gn = xa, sia, dia, ea, ga
                        j = 2 * jj + 1

                    @pl.when(j + 1 < nchunks)
                    def _():
                        basen = tid * per + (j + 1) * 128
                        pltpu.sync_copy(src_hbm.at[pl.ds(basen, 128)], sin_)
                        pltpu.sync_copy(dst_hbm.at[pl.ds(basen, 128)],
                                        din.at[0])
                        pltpu.sync_copy(et_hbm.at[h, pl.ds(basen, 128)], en)
                        pltpu.async_copy(nf_hbm.at[sin_], xn, gn)

                    pltpu.make_async_copy(nf_hbm.at[sic], xc, gc).wait()
                    cc = c_v[...]
                    for q in range(8):
                        sl = pl.ds(q * 16, 16)
                        p_v[sl] = jnp.exp(ec[sl] - cc)

                    @pl.loop(0, 128)
                    def _edge(i):
                        ps = plsc.load_gather(
                            p_v, [jnp.full((16,), i, jnp.int32)])
                        xr = xc.at[i]
                        for q in range(8):
                            sl = pl.ds(q * 16, 16)
                            xr[sl] = xr[sl] * ps

                    pltpu.async_copy(p_v, ssp.at[dic.at[0]], ssem, add=True)
                    pltpu.async_copy(xc, gsp.at[dic.at[0]], ssem, add=True)
                    pltpu.make_async_copy(p_v, ssp.at[dic.at[0]], ssem).wait()
                    pltpu.make_async_copy(xc, gsp.at[dic.at[0]], ssem).wait()

        plsc.subcore_barrier()
        pltpu.sync_copy(
            gsp.at[pl.ds(tid * rows_main, rows_main)],
            g_hbm.at[h, pl.ds(tid * rows_main, rows_main)])

        @pl.when(tid == 15)
        def _():
            pltpu.sync_copy(gsp.at[pl.ds(16 * rows_main, rows_extra)],
                            g_hbm.at[h, pl.ds(16 * rows_main, rows_extra)])

        @pl.when(tid == 0)
        def _():
            pltpu.sync_copy(ssp, s_hbm.at[pl.ds(h * sp, sp)])

        plsc.subcore_barrier()


def _k7_body(heads, out_n, g_ref, s_ref, wn_ref, rn_ref):
    g = g_ref[...]
    s = s_ref[...]
    wn = wn_ref[...]
    acc = jnp.zeros(rn_ref.shape, F32)
    for h in range(heads):
        sh = s[h]
        inv = jnp.where(sh > 0, 1.0 / sh, 0.0)[:, None]
        acc = acc + jnp.dot(g[h] * inv, wn[:, h * out_n:(h + 1) * out_n],
                            preferred_element_type=F32)
    rn_ref[...] = (1.0 / heads) * acc


def kernel(nfeats, efeats, edge_index, W_ni, W_nj, W_fij, W_node, attn, bias):
    N, IN_N = nfeats.shape
    E, IN_E = efeats.shape
    H = attn.shape[1]
    OUT_E = attn.shape[2]
    OUT_N = W_node.shape[1] // H
    HO = H * OUT_E
    E_pad = ((E + 4095) // 4096) * 4096
    pad = E_pad - E

    src_p = jnp.pad(edge_index[0], (0, pad))
    dst_p = jnp.pad(edge_index[1], (0, pad))
    ef_p = jnp.pad(efeats, ((0, pad), (0, 0)))

    Ablk = (attn[0][:, :, None] * jnp.eye(H, dtype=F32)[:, None, :])
    Ablk = Ablk.reshape(HO, H)
    Ablk = jnp.pad(Ablk, ((0, 0), (0, 8 - H)))
    Mmean = jnp.tile(jnp.eye(OUT_E, dtype=F32), (H, 1)) * (1.0 / H)
    bias2 = bias.reshape(1, HO).astype(F32)

    Wcat = jnp.concatenate([W_ni, W_nj], axis=1)
    BN1 = 2000
    fcat = pl.pallas_call(
        _proj_body,
        grid=(N // BN1,),
        in_specs=[
            pl.BlockSpec((BN1, IN_N), lambda i: (i, 0)),
            pl.BlockSpec((IN_N, 2 * HO), lambda i: (0, 0)),
        ],
        out_specs=pl.BlockSpec((BN1, 2 * HO), lambda i: (i, 0)),
        out_shape=jax.ShapeDtypeStruct((N, 2 * HO), F32),
    )(nfeats, Wcat)

    mesh = plsc.VectorSubcoreMesh(core_axis_name="c", subcore_axis_name="s")
    fsum = pl.kernel(
        functools.partial(_k2_body, E_pad, HO),
        out_type=jax.ShapeDtypeStruct((E_pad, 2 * HO), F32),
        mesh=mesh,
        scratch_types=[
            pltpu.VMEM((128,), jnp.int32),
            pltpu.VMEM((128, 2 * HO), F32),
            pltpu.VMEM((128, 2 * HO), F32),
        ],
    )(fcat, src_p, dst_p)

    BE = 2048
    re_p, et, Carr = pl.pallas_call(
        functools.partial(_k3_body, BE, E),
        grid=(E_pad // BE,),
        in_specs=[
            pl.BlockSpec((BE, 2 * HO), lambda i: (i, 0)),
            pl.BlockSpec((BE, IN_E), lambda i: (i, 0)),
            pl.BlockSpec((IN_E, HO), lambda i: (0, 0)),
            pl.BlockSpec((HO, 8), lambda i: (0, 0)),
            pl.BlockSpec((HO, OUT_E), lambda i: (0, 0)),
            pl.BlockSpec((1, HO), lambda i: (0, 0)),
        ],
        out_specs=(
            pl.BlockSpec((BE, OUT_E), lambda i: (i, 0)),
            pl.BlockSpec((8, BE), lambda i: (0, i)),
            pl.BlockSpec((8, 128), lambda i: (0, 0)),
        ),
        out_shape=(
            jax.ShapeDtypeStruct((E_pad, OUT_E), F32),
            jax.ShapeDtypeStruct((8, E_pad), F32),
            jax.ShapeDtypeStruct((8, 128), F32),
        ),
    )(fsum, ef_p, W_fij, Ablk, Mmean, bias2)

    SP = ((N + 639) // 640) * 640
    g, s = pl.kernel(
        functools.partial(_k6_body, E_pad, N, IN_N),
        out_type=(
            jax.ShapeDtypeStruct((H, N, IN_N), F32),
            jax.ShapeDtypeStruct((H * SP,), F32),
        ),
        mesh=plsc.VectorSubcoreMesh(core_axis_name="c", subcore_axis_name="s"),
        compiler_params=_SC_PARAMS,
        scratch_types=[
            pltpu.VMEM((128,), jnp.int32),
            pltpu.VMEM((1, 128), jnp.int32),
            pltpu.VMEM((128, IN_N), F32),
            pltpu.VMEM((128,), F32),
            pltpu.VMEM((128,), F32),
            pltpu.VMEM((16,), F32),
            pltpu.VMEM((SP // 16,), F32),
            pltpu.VMEM((128,), jnp.int32),
            pltpu.VMEM((1, 128), jnp.int32),
            pltpu.VMEM((128, IN_N), F32),
            pltpu.VMEM((128,), F32),
            pltpu.SemaphoreType.DMA,
            pltpu.SemaphoreType.DMA,
            pltpu.SemaphoreType.DMA,
            pltpu.VMEM_SHARED((N, IN_N), F32),
            pltpu.VMEM_SHARED((SP,), F32),
        ],
    )(nfeats, src_p, dst_p, et, Carr)

    s2 = s.reshape(H, SP)
    BN7 = 2048
    rn = pl.pallas_call(
        functools.partial(_k7_body, H, OUT_N),
        grid=((N + BN7 - 1) // BN7,),
        in_specs=[
            pl.BlockSpec((H, BN7, IN_N), lambda i: (0, i, 0)),
            pl.BlockSpec((H, BN7), lambda i: (0, i)),
            pl.BlockSpec((IN_N, H * OUT_N), lambda i: (0, 0)),
        ],
        out_specs=pl.BlockSpec((BN7, OUT_N), lambda i: (i, 0)),
        out_shape=jax.ShapeDtypeStruct((N, OUT_N), F32),
    )(g, s2, W_node)

    return rn, re_p[:E]
